# baseline (speedup 1.0000x reference)
import jax
import jax.numpy as jnp
from jax.experimental import pallas as pl
from jax.experimental.pallas import tpu as pltpu

B, N, D, L = 4, 2048, 32, 64
WIN = 16
INT_MIN = -2147483648
INT_MAX = 2147483647


def _uniform_noise_bits(n):
    key = jax.random.key(42)
    return jax.random.uniform(key, (B, n, n), dtype=jnp.float32)


def _leaky(x):
    return jnp.where(x >= 0, x, 0.01 * x)


def _prep_kernel(x_ref, wip_ref, bip_ref, wd_ref, wk1_ref, bk1_ref,
                 wk2_ref, bk2_ref, wkp_ref, bkp_ref, u_ref, v_ref, k_ref):
    x = x_ref[...]
    dot = lambda a, b: jax.lax.dot(a, b, precision=jax.lax.Precision.DEFAULT,
                                   preferred_element_type=jnp.float32)
    a = _leaky(dot(x, wip_ref[...]) + bip_ref[...])
    m = jnp.max(a, axis=-1, keepdims=True)
    e = jnp.exp(a - m)
    xp = e / jnp.sum(e, axis=-1, keepdims=True)
    u_ref[...] = dot(xp, wd_ref[:L, :])
    v_ref[...] = dot(xp, wd_ref[L:, :])
    h = jnp.maximum(dot(x, wk1_ref[...]) + bk1_ref[...], 0.0)
    mu = dot(h, wk2_ref[...]) + bk2_ref[...]
    k_ref[...] = dot(mu, wkp_ref[...]) + bkp_ref[...]


def _main_kernel(u_ref, v_ref, k_ref, bd_ref, temp_ref, noise_ref, adj_ref):
    n = v_ref.shape[-1]
    br = u_ref.shape[1]
    blk = pl.program_id(1)
    u = u_ref[0]
    v = v_ref[0]
    kk = k_ref[0]
    bd = bd_ref[0]
    temp = temp_ref[0]

    l = _leaky(u + v + bd)
    m = jnp.max(l, axis=1, keepdims=True)
    s = jnp.sum(jnp.exp(l - m), axis=1, keepdims=True)
    g = -jnp.log(-jnp.log(noise_ref[0] + 1e-20) + 1e-20)
    rid = jax.lax.broadcasted_iota(jnp.int32, (br, 1), 0) + blk * br
    cid = jax.lax.broadcasted_iota(jnp.int32, (1, n), 1)
    noise = jnp.where(rid == cid, 0.0, g)
    y = l - m - jnp.log(s) + noise

    bits = jax.lax.bitcast_convert_type(y, jnp.int32)
    mapped = jnp.where(bits >= 0, bits, bits ^ jnp.int32(0x7FFFFFFF))

    c = (1.0 - kk) * (n * 0.5)
    cf = jnp.clip(jnp.floor(c) - 7.0, 0.0, float(n - 1))
    C = cf.astype(jnp.int32)

    def bis_body(i, carry):
        T, cb = carry
        bit = 31 - i
        cand = T + jnp.left_shift(jnp.int32(1), bit)
        cnt = jnp.sum((mapped < cand).astype(jnp.int32), axis=1, keepdims=True)
        ok = cnt <= C
        return jnp.where(ok, cand, T), jnp.where(ok, cnt, cb)

    T0 = jnp.full((br, 1), INT_MIN, jnp.int32)
    T, C_base = jax.lax.fori_loop(0, 32, bis_body,
                                  (T0, jnp.zeros((br, 1), jnp.int32)))

    col32 = jax.lax.broadcasted_iota(jnp.int32, (br, WIN), 1)
    work0 = jnp.where(mapped >= T, mapped, INT_MAX)

    def ext_body(r, carry):
        work, stats = carry
        vmin = jnp.min(work, axis=1, keepdims=True)
        stats = jnp.where(col32 == r, vmin, stats)
        work = jnp.where(work == vmin, INT_MAX, work)
        return work, stats

    stats0 = jnp.full((br, WIN), INT_MAX, jnp.int32)
    _, stats = jax.lax.fori_loop(0, WIN, ext_body, (work0, stats0))

    mtab = jax.lax.broadcasted_iota(jnp.int32, (br, WIN + 1), 1)
    rtab = (C_base - jnp.int32(1)) + mtab
    ttab = rtab.astype(jnp.float32) / float(n) * 2.0 - 1.0
    Ftab = 1.0 + jnp.tanh((ttab + kk) / jnp.float32(0.001))
    f = jnp.zeros((br, n), jnp.float32) + Ftab[:, 0:1]
    for r in range(WIN):
        dF = Ftab[:, r + 1:r + 2] - Ftab[:, r:r + 1]
        f = f + jnp.where(mapped >= stats[:, r:r + 1], dF, 0.0)
    z = (y * f) / temp
    mz = jnp.max(z, axis=1, keepdims=True)
    p = jnp.exp(z - mz)
    adj_ref[0] = p / jnp.sum(p, axis=1, keepdims=True)


def kernel(x, W_ip, b_ip, W_d, b_d, W_k1, b_k1, W_k2, b_k2, W_kp, b_kp, temp):
    n = x.shape[1]
    noise = _uniform_noise_bits(n)
    R = B * n
    CH = 1024

    xf = x.reshape(R, D)
    u, v, k = pl.pallas_call(
        _prep_kernel,
        grid=(R // CH,),
        in_specs=[
            pl.BlockSpec((CH, D), lambda i: (i, 0)),
            pl.BlockSpec((D, L), lambda i: (0, 0)),
            pl.BlockSpec((1, L), lambda i: (0, 0)),
            pl.BlockSpec((2 * L, 1), lambda i: (0, 0)),
            pl.BlockSpec((D, L), lambda i: (0, 0)),
            pl.BlockSpec((1, L), lambda i: (0, 0)),
            pl.BlockSpec((L, L), lambda i: (0, 0)),
            pl.BlockSpec((1, L), lambda i: (0, 0)),
            pl.BlockSpec((L, 1), lambda i: (0, 0)),
            pl.BlockSpec((1, 1), lambda i: (0, 0)),
        ],
        out_specs=[
            pl.BlockSpec((CH, 1), lambda i: (i, 0)),
            pl.BlockSpec((CH, 1), lambda i: (i, 0)),
            pl.BlockSpec((CH, 1), lambda i: (i, 0)),
        ],
        out_shape=[jax.ShapeDtypeStruct((R, 1), jnp.float32)] * 3,
        compiler_params=pltpu.CompilerParams(
            dimension_semantics=("parallel",)),
    )(xf, W_ip, b_ip.reshape(1, L), W_d, W_k1, b_k1.reshape(1, L),
      W_k2, b_k2.reshape(1, L), W_kp, b_kp.reshape(1, 1))

    BR = 512
    adj = pl.pallas_call(
        _main_kernel,
        grid=(B, n // BR),
        in_specs=[
            pl.BlockSpec((1, BR, 1), lambda b, i: (b, i, 0)),
            pl.BlockSpec((1, 1, n), lambda b, i: (b, 0, 0)),
            pl.BlockSpec((1, BR, 1), lambda b, i: (b, i, 0)),
            pl.BlockSpec(memory_space=pltpu.SMEM),
            pl.BlockSpec(memory_space=pltpu.SMEM),
            pl.BlockSpec((1, BR, n), lambda b, i: (b, i, 0)),
        ],
        out_specs=pl.BlockSpec((1, BR, n), lambda b, i: (b, i, 0)),
        out_shape=jax.ShapeDtypeStruct((B, n, n), jnp.float32),
        compiler_params=pltpu.CompilerParams(
            dimension_semantics=("parallel", "parallel")),
    )(u.reshape(B, n, 1), v.reshape(B, 1, n), k.reshape(B, n, 1),
      b_d, jnp.asarray(temp, jnp.float32).reshape(1), noise)

    return adj, k.reshape(B, n, 1)


# speedup vs baseline: 1.1358x; 1.1358x over previous
import jax
import jax.numpy as jnp
from jax.experimental import pallas as pl
from jax.experimental.pallas import tpu as pltpu

B, N, D, L = 4, 2048, 32, 64
WIN = 12
INT_MIN = -2147483648
INT_MAX = 2147483647


def _gumbel_noise(n):
    key = jax.random.key(42)
    U = jax.random.uniform(key, (B, n, n), dtype=jnp.float32)
    g = -jnp.log(-jnp.log(U + 1e-20) + 1e-20)
    zero_self = 1.0 - jnp.eye(n, dtype=jnp.float32)
    return g * zero_self[None, :, :]


def _leaky(x):
    return jnp.where(x >= 0, x, 0.01 * x)


def _prep_kernel(x_ref, wip_ref, bip_ref, wd_ref, wk1_ref, bk1_ref,
                 wk2_ref, bk2_ref, wkp_ref, bkp_ref, u_ref, v_ref, k_ref):
    x = x_ref[...]
    dot = lambda a, b: jax.lax.dot(a, b, precision=jax.lax.Precision.DEFAULT,
                                   preferred_element_type=jnp.float32)
    a = _leaky(dot(x, wip_ref[...]) + bip_ref[...])
    m = jnp.max(a, axis=-1, keepdims=True)
    e = jnp.exp(a - m)
    xp = e / jnp.sum(e, axis=-1, keepdims=True)
    u_ref[...] = dot(xp, wd_ref[:L, :])
    v_ref[...] = dot(xp, wd_ref[L:, :])
    h = jnp.maximum(dot(x, wk1_ref[...]) + bk1_ref[...], 0.0)
    mu = dot(h, wk2_ref[...]) + bk2_ref[...]
    k_ref[...] = dot(mu, wkp_ref[...]) + bkp_ref[...]


def _main_kernel(u_ref, v_ref, k_ref, bd_ref, temp_ref, noise_ref, adj_ref):
    n = v_ref.shape[-1]
    br = u_ref.shape[1]
    u = u_ref[0]
    v = v_ref[0]
    kk = k_ref[0]
    bd = bd_ref[0]
    temp = temp_ref[0]

    l = _leaky(u + v + bd)
    m = jnp.max(l, axis=1, keepdims=True)
    s = jnp.sum(jnp.exp(l - m), axis=1, keepdims=True)
    y = l - m - jnp.log(s) + noise_ref[0]

    bits = jax.lax.bitcast_convert_type(y, jnp.int32)
    mapped = jnp.where(bits >= 0, bits, bits ^ jnp.int32(0x7FFFFFFF))

    c = (1.0 - kk) * (n * 0.5)
    cf = jnp.clip(jnp.floor(c) - 5.0, 0.0, float(n - 1))
    C = cf.astype(jnp.int32)

    def bis_body(i, carry):
        T, cb = carry
        bit = 31 - i
        cand = T + jnp.left_shift(jnp.int32(1), bit)
        cnt = jnp.sum((mapped < cand).astype(jnp.int32), axis=1, keepdims=True)
        ok = cnt <= C
        return jnp.where(ok, cand, T), jnp.where(ok, cnt, cb)

    T0 = jnp.full((br, 1), INT_MIN, jnp.int32)
    T, C_base = jax.lax.fori_loop(0, 32, bis_body,
                                  (T0, jnp.zeros((br, 1), jnp.int32)))

    col32 = jax.lax.broadcasted_iota(jnp.int32, (br, WIN), 1)
    work0 = jnp.where(mapped >= T, mapped, INT_MAX)

    def ext_body(r, carry):
        work, stats = carry
        vmin = jnp.min(work, axis=1, keepdims=True)
        stats = jnp.where(col32 == r, vmin, stats)
        work = jnp.where(work == vmin, INT_MAX, work)
        return work, stats

    stats0 = jnp.full((br, WIN), INT_MAX, jnp.int32)
    _, stats = jax.lax.fori_loop(0, WIN, ext_body, (work0, stats0))

    mtab = jax.lax.broadcasted_iota(jnp.int32, (br, WIN + 1), 1)
    rtab = (C_base - jnp.int32(1)) + mtab
    ttab = rtab.astype(jnp.float32) / float(n) * 2.0 - 1.0
    Ftab = 1.0 + jnp.tanh((ttab + kk) / jnp.float32(0.001))
    f = jnp.zeros((br, n), jnp.float32) + Ftab[:, 0:1]
    for r in range(WIN):
        dF = Ftab[:, r + 1:r + 2] - Ftab[:, r:r + 1]
        f = f + jnp.where(mapped >= stats[:, r:r + 1], dF, 0.0)
    z = (y * f) / temp
    mz = jnp.max(z, axis=1, keepdims=True)
    p = jnp.exp(z - mz)
    adj_ref[0] = p / jnp.sum(p, axis=1, keepdims=True)


def kernel(x, W_ip, b_ip, W_d, b_d, W_k1, b_k1, W_k2, b_k2, W_kp, b_kp, temp):
    n = x.shape[1]
    noise = _gumbel_noise(n)
    R = B * n
    CH = 1024

    xf = x.reshape(R, D)
    u, v, k = pl.pallas_call(
        _prep_kernel,
        grid=(R // CH,),
        in_specs=[
            pl.BlockSpec((CH, D), lambda i: (i, 0)),
            pl.BlockSpec((D, L), lambda i: (0, 0)),
            pl.BlockSpec((1, L), lambda i: (0, 0)),
            pl.BlockSpec((2 * L, 1), lambda i: (0, 0)),
            pl.BlockSpec((D, L), lambda i: (0, 0)),
            pl.BlockSpec((1, L), lambda i: (0, 0)),
            pl.BlockSpec((L, L), lambda i: (0, 0)),
            pl.BlockSpec((1, L), lambda i: (0, 0)),
            pl.BlockSpec((L, 1), lambda i: (0, 0)),
            pl.BlockSpec((1, 1), lambda i: (0, 0)),
        ],
        out_specs=[
            pl.BlockSpec((CH, 1), lambda i: (i, 0)),
            pl.BlockSpec((CH, 1), lambda i: (i, 0)),
            pl.BlockSpec((CH, 1), lambda i: (i, 0)),
        ],
        out_shape=[jax.ShapeDtypeStruct((R, 1), jnp.float32)] * 3,
        compiler_params=pltpu.CompilerParams(
            dimension_semantics=("parallel",)),
    )(xf, W_ip, b_ip.reshape(1, L), W_d, W_k1, b_k1.reshape(1, L),
      W_k2, b_k2.reshape(1, L), W_kp, b_kp.reshape(1, 1))

    BR = 512
    adj = pl.pallas_call(
        _main_kernel,
        grid=(B, n // BR),
        in_specs=[
            pl.BlockSpec((1, BR, 1), lambda b, i: (b, i, 0)),
            pl.BlockSpec((1, 1, n), lambda b, i: (b, 0, 0)),
            pl.BlockSpec((1, BR, 1), lambda b, i: (b, i, 0)),
            pl.BlockSpec(memory_space=pltpu.SMEM),
            pl.BlockSpec(memory_space=pltpu.SMEM),
            pl.BlockSpec((1, BR, n), lambda b, i: (b, i, 0)),
        ],
        out_specs=pl.BlockSpec((1, BR, n), lambda b, i: (b, i, 0)),
        out_shape=jax.ShapeDtypeStruct((B, n, n), jnp.float32),
        compiler_params=pltpu.CompilerParams(
            dimension_semantics=("parallel", "parallel")),
    )(u.reshape(B, n, 1), v.reshape(B, 1, n), k.reshape(B, n, 1),
      b_d, jnp.asarray(temp, jnp.float32).reshape(1), noise)

    return adj, k.reshape(B, n, 1)
